# Initial kernel scaffold; baseline (speedup 1.0000x reference)
import functools

import jax
import jax.numpy as jnp
from jax import lax
from jax.experimental import pallas as pl
from jax.experimental.pallas import tpu as pltpu
from jax.experimental.pallas import tpu_sc as plsc


def _make_gather(V, D, NW, NCHUNK, C):
    mesh = plsc.VectorSubcoreMesh(core_axis_name="c", subcore_axis_name="s")
    NC = 2
    b_per_w = NCHUNK * C

    @functools.partial(
        pl.kernel,
        out_type=jax.ShapeDtypeStruct((NW * NCHUNK * C, D), jnp.float32),
        mesh=mesh,
        scratch_types=[
            pltpu.VMEM((NCHUNK, C), jnp.int32),
            pltpu.VMEM((C, D), jnp.float32),
            pltpu.SemaphoreType.DMA,
        ],
    )
    def gather_kernel(table_hbm, idx_hbm, out_hbm, idx_v, rows_v, sem):
        wid = lax.axis_index("s") * NC + lax.axis_index("c")
        base = wid * b_per_w
        pltpu.sync_copy(idx_hbm.at[wid], idx_v)

        def body(j, carry):
            pltpu.async_copy(table_hbm.at[idx_v.at[j]], rows_v, sem).wait()
            pltpu.sync_copy(rows_v, out_hbm.at[pl.ds(base + j * C, C)])
            return carry

        lax.fori_loop(0, NCHUNK, body, 0, unroll=False)

    return gather_kernel


def kernel(input_nodes, table):
    B0, B1 = input_nodes.shape
    V, D = table.shape
    B = B0 * B1
    NW = 32
    C = 1280
    NCHUNK = B // (NW * C)
    assert NW * NCHUNK * C == B
    idx = input_nodes.reshape(NW, NCHUNK, C).astype(jnp.int32)
    out = _make_gather(V, D, NW, NCHUNK, C)(table, idx)
    return out.reshape(B0, B1, D)


# speedup vs baseline: 1.0995x; 1.0995x over previous
import functools

import jax
import jax.numpy as jnp
from jax import lax
from jax.experimental import pallas as pl
from jax.experimental.pallas import tpu as pltpu
from jax.experimental.pallas import tpu_sc as plsc


def _make_gather(V, D, NW, NCHUNK, C):
    mesh = plsc.VectorSubcoreMesh(core_axis_name="c", subcore_axis_name="s")
    NC = 2
    b_per_w = NCHUNK * C

    @functools.partial(
        pl.kernel,
        out_type=jax.ShapeDtypeStruct((NW * NCHUNK * C, D), jnp.float32),
        mesh=mesh,
        scratch_types=[
            pltpu.VMEM((C,), jnp.int32),
            pltpu.VMEM((C, D), jnp.float32),
            pltpu.SemaphoreType.DMA,
        ],
        compiler_params=pltpu.CompilerParams(use_tc_tiling_on_sc=False),
    )
    def gather_kernel(table_hbm, idx_hbm, out_hbm, idx_v, rows_v, sem):
        wid = lax.axis_index("s") * NC + lax.axis_index("c")
        base = wid * b_per_w

        def body(j, carry):
            pltpu.sync_copy(idx_hbm.at[wid, j], idx_v)
            pltpu.async_copy(table_hbm.at[idx_v], rows_v, sem).wait()
            pltpu.sync_copy(rows_v, out_hbm.at[pl.ds(base + j * C, C)])
            return carry

        lax.fori_loop(0, NCHUNK, body, 0, unroll=False)

    return gather_kernel


def kernel(input_nodes, table):
    B0, B1 = input_nodes.shape
    V, D = table.shape
    B = B0 * B1
    NW = 32
    C = 1280
    NCHUNK = B // (NW * C)
    assert NW * NCHUNK * C == B
    idx = input_nodes.reshape(NW, NCHUNK, C).astype(jnp.int32)
    out = _make_gather(V, D, NW, NCHUNK, C)(table, idx)
    return out.reshape(B0, B1, D)


# speedup vs baseline: 1.1101x; 1.0096x over previous
import functools

import jax
import jax.numpy as jnp
from jax import lax
from jax.experimental import pallas as pl
from jax.experimental.pallas import tpu as pltpu
from jax.experimental.pallas import tpu_sc as plsc


def _make_gather(V, D, NW, NB, C, NCHUNK):
    mesh = plsc.VectorSubcoreMesh(core_axis_name="c", subcore_axis_name="s")
    NC = 2
    b_per_w = NCHUNK * C
    NG = NCHUNK // NB
    assert NG * NB == NCHUNK

    @functools.partial(
        pl.kernel,
        out_type=jax.ShapeDtypeStruct((NW * b_per_w, D), jnp.float32),
        mesh=mesh,
        scratch_types=[
            pltpu.VMEM((NCHUNK, C), jnp.int32),
            *[pltpu.VMEM((C, D), jnp.float32) for _ in range(NB)],
            *[pltpu.SemaphoreType.DMA for _ in range(2 * NB)],
        ],
        compiler_params=pltpu.CompilerParams(use_tc_tiling_on_sc=False),
    )
    def gather_kernel(table_hbm, idx_hbm, out_hbm, idx_v, *bufs):
        rows = bufs[:NB]
        semg = bufs[NB : 2 * NB]
        semo = bufs[2 * NB : 3 * NB]
        wid = lax.axis_index("s") * NC + lax.axis_index("c")
        base = wid * b_per_w
        pltpu.sync_copy(idx_hbm.at[wid], idx_v)

        for b in range(NB):
            pltpu.async_copy(table_hbm.at[idx_v.at[b]], rows[b], semg[b])

        def body(g, carry):
            for b in range(NB):
                j = g * NB + b
                pltpu.make_async_copy(table_hbm.at[idx_v.at[b]], rows[b], semg[b]).wait()
                pltpu.async_copy(rows[b], out_hbm.at[pl.ds(base + j * C, C)], semo[b])

            @pl.when(g < NG - 1)
            def _():
                for b in range(NB):
                    jn = (g + 1) * NB + b
                    pltpu.make_async_copy(rows[b], out_hbm.at[pl.ds(0, C)], semo[b]).wait()
                    pltpu.async_copy(table_hbm.at[idx_v.at[jn]], rows[b], semg[b])

            return carry

        lax.fori_loop(0, NG, body, 0, unroll=False)
        for b in range(NB):
            pltpu.make_async_copy(rows[b], out_hbm.at[pl.ds(0, C)], semo[b]).wait()

    return gather_kernel


def kernel(input_nodes, table):
    B0, B1 = input_nodes.shape
    V, D = table.shape
    B = B0 * B1
    NW = 32
    NB = 4
    C = 640
    NCHUNK = B // (NW * C)
    assert NW * NCHUNK * C == B
    idx = input_nodes.reshape(NW, NCHUNK, C).astype(jnp.int32)
    out = _make_gather(V, D, NW, NB, C, NCHUNK)(table, idx)
    return out.reshape(B0, B1, D)


# speedup vs baseline: 1.8081x; 1.6288x over previous
import functools

import jax
import jax.numpy as jnp
from jax import lax
from jax.experimental import pallas as pl
from jax.experimental.pallas import tpu as pltpu
from jax.experimental.pallas import tpu_sc as plsc


def _make_gather(V, D, B0, B1, NW, NB, G):
    mesh = plsc.VectorSubcoreMesh(core_axis_name="c", subcore_axis_name="s")
    NC = 2
    rows_per_w = B0 // NW
    NGRP = rows_per_w // G
    NSUP = NGRP // NB
    assert NSUP * NB * G == rows_per_w

    @functools.partial(
        pl.kernel,
        out_type=jax.ShapeDtypeStruct((B0, B1, D), jnp.float32),
        mesh=mesh,
        scratch_types=[
            pltpu.VMEM((rows_per_w, B1), jnp.int32),
            *[pltpu.VMEM((G, B1, D), jnp.float32) for _ in range(NB)],
            *[pltpu.SemaphoreType.DMA for _ in range(2 * NB)],
        ],
        compiler_params=pltpu.CompilerParams(use_tc_tiling_on_sc=False),
    )
    def gather_kernel(table_hbm, idx_hbm, out_hbm, idx_v, *bufs):
        rows = bufs[:NB]
        semg = bufs[NB : 2 * NB]
        semo = bufs[2 * NB : 3 * NB]
        wid = lax.axis_index("s") * NC + lax.axis_index("c")
        base = wid * rows_per_w
        pltpu.sync_copy(idx_hbm.at[pl.ds(base, rows_per_w)], idx_v)

        def fill(b, grp):
            for r in range(G):
                pltpu.async_copy(
                    table_hbm.at[idx_v.at[grp * G + r]], rows[b].at[r], semg[b]
                )

        def drain_fill(b):
            for r in range(G):
                pltpu.make_async_copy(
                    table_hbm.at[idx_v.at[r]], rows[b].at[r], semg[b]
                ).wait()

        for b in range(NB):
            fill(b, b)

        def body(s, carry):
            for b in range(NB):
                grp = s * NB + b
                drain_fill(b)
                pltpu.async_copy(
                    rows[b], out_hbm.at[pl.ds(base + grp * G, G)], semo[b]
                )

            @pl.when(s < NSUP - 1)
            def _():
                for b in range(NB):
                    pltpu.make_async_copy(
                        rows[b], out_hbm.at[pl.ds(0, G)], semo[b]
                    ).wait()
                    fill(b, (s + 1) * NB + b)

            return carry

        lax.fori_loop(0, NSUP, body, 0, unroll=False)
        for b in range(NB):
            pltpu.make_async_copy(rows[b], out_hbm.at[pl.ds(0, G)], semo[b]).wait()

    return gather_kernel


def kernel(input_nodes, table):
    B0, B1 = input_nodes.shape
    V, D = table.shape
    idx = input_nodes.astype(jnp.int32)
    return _make_gather(V, D, B0, B1, NW=32, NB=4, G=8)(table, idx)
